# Initial kernel scaffold; baseline (speedup 1.0000x reference)
import functools

import jax
import jax.numpy as jnp
from jax import lax
from jax.experimental import pallas as pl
from jax.experimental.pallas import tpu as pltpu
from jax.experimental.pallas import tpu_sc as plsc

_NC = 2
_NS = 16
_NW = _NC * _NS
_IB = 128
_BN = 1024


def _sc_mesh():
    return plsc.VectorSubcoreMesh(core_axis_name="c", subcore_axis_name="s")


def _deg_kernel(acc_n, eblocks):
    bpt = eblocks // _NW
    nch = bpt // 16
    rpt = acc_n // _NS

    @functools.partial(
        pl.kernel,
        out_type=jax.ShapeDtypeStruct((_NC * acc_n,), jnp.float32),
        mesh=_sc_mesh(),
        scratch_types=[
            pltpu.VMEM((16, _IB), jnp.int32),
            pltpu.VMEM((_IB,), jnp.float32),
            pltpu.VMEM_SHARED((acc_n,), jnp.float32),
        ],
    )
    def deg(dst_hbm, ones_hbm, zeros_hbm, out_hbm, idx, ones, dacc):
        c = lax.axis_index("c")
        s = lax.axis_index("s")
        pltpu.sync_copy(zeros_hbm.at[pl.ds(s * rpt, rpt)],
                        dacc.at[pl.ds(s * rpt, rpt)])
        pltpu.sync_copy(ones_hbm, ones)
        plsc.subcore_barrier()
        base = (c * _NS + s) * bpt

        def body(i, carry):
            pltpu.sync_copy(dst_hbm.at[pl.ds(base + i * 16, 16)], idx)
            for j in range(16):
                pltpu.sync_copy(ones, dacc.at[idx.at[j]], add=True)
            return carry

        lax.fori_loop(0, nch, body, 0)
        plsc.subcore_barrier()
        pltpu.sync_copy(dacc.at[pl.ds(s * rpt, rpt)],
                        out_hbm.at[pl.ds(c * acc_n + s * rpt, rpt)])

    return deg


def _agg_kernel(acc_n, eblocks, feat):
    bpt = eblocks // _NW
    nch = bpt // 16
    rpt = acc_n // _NS

    @functools.partial(
        pl.kernel,
        out_type=jax.ShapeDtypeStruct((_NC * acc_n, feat), jnp.float32),
        mesh=_sc_mesh(),
        scratch_types=[
            pltpu.VMEM((16, _IB), jnp.int32),
            pltpu.VMEM((16, _IB), jnp.int32),
            pltpu.VMEM((16 * _IB, feat), jnp.float32),
            pltpu.VMEM_SHARED((acc_n, feat), jnp.float32),
            pltpu.SemaphoreType.DMA,
        ],
    )
    def agg(g_hbm, src_hbm, dst_hbm, out_hbm, sidx, didx, rows, acc, sem):
        c = lax.axis_index("c")
        s = lax.axis_index("s")
        pltpu.sync_copy(g_hbm.at[pl.ds(s * rpt, rpt)],
                        acc.at[pl.ds(s * rpt, rpt)])
        plsc.subcore_barrier()
        base = (c * _NS + s) * bpt

        def body(i, carry):
            pltpu.sync_copy(src_hbm.at[pl.ds(base + i * 16, 16)], sidx)
            pltpu.sync_copy(dst_hbm.at[pl.ds(base + i * 16, 16)], didx)
            copies = []
            for j in range(16):
                copies.append(pltpu.async_copy(
                    g_hbm.at[sidx.at[j]], rows.at[pl.ds(j * _IB, _IB)], sem))
            for j in range(16):
                copies[j].wait()
                pltpu.sync_copy(rows.at[pl.ds(j * _IB, _IB)],
                                acc.at[didx.at[j]], add=True)
            return carry

        lax.fori_loop(0, nch, body, 0)
        plsc.subcore_barrier()
        pltpu.sync_copy(acc.at[pl.ds(s * rpt, rpt)],
                        out_hbm.at[pl.ds(c * acc_n + s * rpt, rpt)])

    return agg


def _mm1_body(x_ref, w_ref, dega_ref, degb_ref, g_ref, dinv_ref):
    deg = dega_ref[...] + degb_ref[...] + 1.0
    dinv = lax.rsqrt(deg)
    hm = jnp.dot(x_ref[...], w_ref[...], preferred_element_type=jnp.float32)
    g_ref[...] = hm * dinv
    dinv_ref[...] = dinv


def _mm1(x, w1, dega, degb, acc_n):
    f_in = x.shape[1]
    h = w1.shape[1]
    return pl.pallas_call(
        _mm1_body,
        grid=(acc_n // _BN,),
        in_specs=[
            pl.BlockSpec((_BN, f_in), lambda i: (i, 0)),
            pl.BlockSpec((f_in, h), lambda i: (0, 0)),
            pl.BlockSpec((_BN, 1), lambda i: (i, 0)),
            pl.BlockSpec((_BN, 1), lambda i: (i, 0)),
        ],
        out_specs=[
            pl.BlockSpec((_BN, h), lambda i: (i, 0)),
            pl.BlockSpec((_BN, 1), lambda i: (i, 0)),
        ],
        out_shape=[
            jax.ShapeDtypeStruct((acc_n, h), jnp.float32),
            jax.ShapeDtypeStruct((acc_n, 1), jnp.float32),
        ],
    )(x, w1, dega, degb)


def _mid_body(a0_ref, a1_ref, g1_ref, dinv_ref, b1_ref, w2_ref, g2_ref):
    sgm = a0_ref[...] + a1_ref[...] - g1_ref[...]
    dinv = dinv_ref[...]
    h2 = jnp.maximum(dinv * sgm + b1_ref[...], 0.0)
    g2_ref[...] = jnp.dot(h2, w2_ref[...],
                          preferred_element_type=jnp.float32) * dinv


def _mid(a0, a1, g1, dinv, b1, w2p, acc_n):
    h = g1.shape[1]
    return pl.pallas_call(
        _mid_body,
        grid=(acc_n // _BN,),
        in_specs=[
            pl.BlockSpec((_BN, h), lambda i: (i, 0)),
            pl.BlockSpec((_BN, h), lambda i: (i, 0)),
            pl.BlockSpec((_BN, h), lambda i: (i, 0)),
            pl.BlockSpec((_BN, 1), lambda i: (i, 0)),
            pl.BlockSpec((1, h), lambda i: (0, 0)),
            pl.BlockSpec((h, h), lambda i: (0, 0)),
        ],
        out_specs=pl.BlockSpec((_BN, h), lambda i: (i, 0)),
        out_shape=jax.ShapeDtypeStruct((acc_n, h), jnp.float32),
    )(a0, a1, g1, dinv, b1, w2p)


def _fin_body(n_cls, a0_ref, a1_ref, g2_ref, dinv_ref, b2_ref, o_ref):
    z = dinv_ref[...] * (a0_ref[...] + a1_ref[...] - g2_ref[...]) + b2_ref[...]
    col = lax.broadcasted_iota(jnp.int32, z.shape, 1)
    valid = col < n_cls
    zm = jnp.where(valid, z, -jnp.inf)
    m = jnp.max(zm, axis=1, keepdims=True)
    e = jnp.where(valid, jnp.exp(z - m), 0.0)
    lse = jnp.log(jnp.sum(e, axis=1, keepdims=True))
    o_ref[...] = (z - m - lse)[:, :n_cls]


def _fin(a0, a1, g2, dinv, b2p, n, n_cls, acc_n):
    h = g2.shape[1]
    return pl.pallas_call(
        functools.partial(_fin_body, n_cls),
        grid=(acc_n // _BN,),
        in_specs=[
            pl.BlockSpec((_BN, h), lambda i: (i, 0)),
            pl.BlockSpec((_BN, h), lambda i: (i, 0)),
            pl.BlockSpec((_BN, h), lambda i: (i, 0)),
            pl.BlockSpec((_BN, 1), lambda i: (i, 0)),
            pl.BlockSpec((1, h), lambda i: (0, 0)),
        ],
        out_specs=pl.BlockSpec((_BN, n_cls), lambda i: (i, 0)),
        out_shape=jax.ShapeDtypeStruct((n, n_cls), jnp.float32),
    )(a0, a1, g2, dinv, b2p)


def kernel(x, edge_index, W1, b1, W2, b2):
    n, _ = x.shape
    e = edge_index.shape[1]
    h = W1.shape[1]
    n_cls = W2.shape[1]
    assert h == 16, "feature width must match one 64B DMA granule"

    acc_n = -(-n // _BN) * _BN
    if acc_n == n:
        acc_n += _BN
    pad_rows = acc_n - n

    step = _NW * 16 * _IB
    e2 = -(-e // step) * step
    eblocks = e2 // _IB
    npad = e2 - e
    ar = jnp.arange(npad, dtype=jnp.int32)
    src2 = jnp.concatenate([edge_index[0], ar % n]).reshape(eblocks, _IB)
    dst2 = jnp.concatenate([edge_index[1], n + (ar % pad_rows)]).reshape(eblocks, _IB)

    ones = jnp.ones((_IB,), jnp.float32)
    zeros = jnp.zeros((acc_n,), jnp.float32)
    w2p = jnp.pad(W2, ((0, 0), (0, h - n_cls)))
    b2p = jnp.pad(b2, (0, h - n_cls)).reshape(1, h)

    degs = _deg_kernel(acc_n, eblocks)(dst2, ones, zeros)
    dega = degs[:acc_n].reshape(acc_n, 1)
    degb = degs[acc_n:].reshape(acc_n, 1)

    g1, dinv = _mm1(x, W1, dega, degb, acc_n)

    agg = _agg_kernel(acc_n, eblocks, h)
    accs1 = agg(g1, src2, dst2)
    g2 = _mid(accs1[:acc_n], accs1[acc_n:], g1, dinv, b1.reshape(1, h), w2p,
              acc_n)
    accs2 = agg(g2, src2, dst2)
    return _fin(accs2[:acc_n], accs2[acc_n:], g2, dinv, b2p, n, n_cls, acc_n)


# speedup vs baseline: 38.5326x; 38.5326x over previous
import functools

import jax
import jax.numpy as jnp
from jax import lax
from jax.experimental import pallas as pl
from jax.experimental.pallas import tpu as pltpu
from jax.experimental.pallas import tpu_sc as plsc

_NC = 2
_NS = 16
_NW = _NC * _NS
_IB = 128
_BN = 1024


def _sc_mesh():
    return plsc.VectorSubcoreMesh(core_axis_name="c", subcore_axis_name="s")


_SC_PARAMS = pltpu.CompilerParams(use_tc_tiling_on_sc=False)


def _deg_kernel(acc_n, eblocks):
    bpt = eblocks // _NW
    nch = bpt // 16
    rpt = acc_n // _NS

    @functools.partial(
        pl.kernel,
        out_type=jax.ShapeDtypeStruct((_NC * acc_n,), jnp.float32),
        mesh=_sc_mesh(),
        compiler_params=_SC_PARAMS,
        scratch_types=[
            pltpu.VMEM((16, _IB), jnp.int32),
            pltpu.VMEM((_IB,), jnp.float32),
            pltpu.VMEM_SHARED((acc_n,), jnp.float32),
        ],
    )
    def deg(dst_hbm, ones_hbm, zeros_hbm, out_hbm, idx, ones, dacc):
        c = lax.axis_index("c")
        s = lax.axis_index("s")
        pltpu.sync_copy(zeros_hbm.at[pl.ds(s * rpt, rpt)],
                        dacc.at[pl.ds(s * rpt, rpt)])
        pltpu.sync_copy(ones_hbm, ones)
        plsc.subcore_barrier()
        base = (c * _NS + s) * bpt

        def body(i, carry):
            pltpu.sync_copy(dst_hbm.at[pl.ds(base + i * 16, 16)], idx)
            for j in range(16):
                pltpu.sync_copy(ones, dacc.at[idx.at[j]], add=True)
            return carry

        lax.fori_loop(0, nch, body, 0)
        plsc.subcore_barrier()
        pltpu.sync_copy(dacc.at[pl.ds(s * rpt, rpt)],
                        out_hbm.at[pl.ds(c * acc_n + s * rpt, rpt)])

    return deg


_CB = 8


def _agg_kernel(acc_n, eblocks, feat):
    bpt = eblocks // _NW
    nch = bpt // _CB
    rpt = acc_n // _NS

    @functools.partial(
        pl.kernel,
        out_type=jax.ShapeDtypeStruct((_NC * acc_n, feat), jnp.float32),
        mesh=_sc_mesh(),
        compiler_params=_SC_PARAMS,
        scratch_types=[
            pltpu.VMEM((_CB, _IB), jnp.int32),
            pltpu.VMEM((_CB, _IB), jnp.int32),
            pltpu.VMEM((_CB * _IB, feat), jnp.float32),
            pltpu.VMEM_SHARED((acc_n, feat), jnp.float32),
            pltpu.SemaphoreType.DMA,
        ],
    )
    def agg(g_hbm, src_hbm, dst_hbm, out_hbm, sidx, didx, rows, acc, sem):
        c = lax.axis_index("c")
        s = lax.axis_index("s")
        pltpu.sync_copy(g_hbm.at[pl.ds(s * rpt, rpt)],
                        acc.at[pl.ds(s * rpt, rpt)])
        plsc.subcore_barrier()
        base = (c * _NS + s) * bpt

        def body(i, carry):
            pltpu.sync_copy(src_hbm.at[pl.ds(base + i * _CB, _CB)], sidx)
            pltpu.sync_copy(dst_hbm.at[pl.ds(base + i * _CB, _CB)], didx)
            copies = []
            for j in range(_CB):
                copies.append(pltpu.async_copy(
                    g_hbm.at[sidx.at[j]], rows.at[pl.ds(j * _IB, _IB)], sem))
            for j in range(_CB):
                copies[j].wait()
                pltpu.sync_copy(rows.at[pl.ds(j * _IB, _IB)],
                                acc.at[didx.at[j]], add=True)
            return carry

        lax.fori_loop(0, nch, body, 0)
        plsc.subcore_barrier()
        pltpu.sync_copy(acc.at[pl.ds(s * rpt, rpt)],
                        out_hbm.at[pl.ds(c * acc_n + s * rpt, rpt)])

    return agg


def _mm1_body(x_ref, w_ref, dega_ref, degb_ref, g_ref, dinv_ref):
    deg = dega_ref[...] + degb_ref[...] + 1.0
    dinv = lax.rsqrt(deg)
    hm = jnp.dot(x_ref[...], w_ref[...], preferred_element_type=jnp.float32)
    g_ref[...] = hm * dinv
    dinv_ref[...] = dinv


def _mm1(x, w1, dega, degb, acc_n):
    f_in = x.shape[1]
    h = w1.shape[1]
    return pl.pallas_call(
        _mm1_body,
        grid=(acc_n // _BN,),
        in_specs=[
            pl.BlockSpec((_BN, f_in), lambda i: (i, 0)),
            pl.BlockSpec((f_in, h), lambda i: (0, 0)),
            pl.BlockSpec((_BN, 1), lambda i: (i, 0)),
            pl.BlockSpec((_BN, 1), lambda i: (i, 0)),
        ],
        out_specs=[
            pl.BlockSpec((_BN, h), lambda i: (i, 0)),
            pl.BlockSpec((_BN, 1), lambda i: (i, 0)),
        ],
        out_shape=[
            jax.ShapeDtypeStruct((acc_n, h), jnp.float32),
            jax.ShapeDtypeStruct((acc_n, 1), jnp.float32),
        ],
    )(x, w1, dega, degb)


def _mid_body(a0_ref, a1_ref, g1_ref, dinv_ref, b1_ref, w2_ref, g2_ref):
    sgm = a0_ref[...] + a1_ref[...] - g1_ref[...]
    dinv = dinv_ref[...]
    h2 = jnp.maximum(dinv * sgm + b1_ref[...], 0.0)
    g2_ref[...] = jnp.dot(h2, w2_ref[...],
                          preferred_element_type=jnp.float32) * dinv


def _mid(a0, a1, g1, dinv, b1, w2p, acc_n):
    h = g1.shape[1]
    return pl.pallas_call(
        _mid_body,
        grid=(acc_n // _BN,),
        in_specs=[
            pl.BlockSpec((_BN, h), lambda i: (i, 0)),
            pl.BlockSpec((_BN, h), lambda i: (i, 0)),
            pl.BlockSpec((_BN, h), lambda i: (i, 0)),
            pl.BlockSpec((_BN, 1), lambda i: (i, 0)),
            pl.BlockSpec((1, h), lambda i: (0, 0)),
            pl.BlockSpec((h, h), lambda i: (0, 0)),
        ],
        out_specs=pl.BlockSpec((_BN, h), lambda i: (i, 0)),
        out_shape=jax.ShapeDtypeStruct((acc_n, h), jnp.float32),
    )(a0, a1, g1, dinv, b1, w2p)


def _fin_body(n_cls, a0_ref, a1_ref, g2_ref, dinv_ref, b2_ref, o_ref):
    z = dinv_ref[...] * (a0_ref[...] + a1_ref[...] - g2_ref[...]) + b2_ref[...]
    col = lax.broadcasted_iota(jnp.int32, z.shape, 1)
    valid = col < n_cls
    zm = jnp.where(valid, z, -jnp.inf)
    m = jnp.max(zm, axis=1, keepdims=True)
    e = jnp.where(valid, jnp.exp(z - m), 0.0)
    lse = jnp.log(jnp.sum(e, axis=1, keepdims=True))
    o_ref[...] = (z - m - lse)[:, :n_cls]


def _fin(a0, a1, g2, dinv, b2p, n, n_cls, acc_n):
    h = g2.shape[1]
    return pl.pallas_call(
        functools.partial(_fin_body, n_cls),
        grid=(acc_n // _BN,),
        in_specs=[
            pl.BlockSpec((_BN, h), lambda i: (i, 0)),
            pl.BlockSpec((_BN, h), lambda i: (i, 0)),
            pl.BlockSpec((_BN, h), lambda i: (i, 0)),
            pl.BlockSpec((_BN, 1), lambda i: (i, 0)),
            pl.BlockSpec((1, h), lambda i: (0, 0)),
        ],
        out_specs=pl.BlockSpec((_BN, n_cls), lambda i: (i, 0)),
        out_shape=jax.ShapeDtypeStruct((n, n_cls), jnp.float32),
    )(a0, a1, g2, dinv, b2p)


def kernel(x, edge_index, W1, b1, W2, b2):
    n, _ = x.shape
    e = edge_index.shape[1]
    h = W1.shape[1]
    n_cls = W2.shape[1]
    assert h == 16, "feature width must match one 64B DMA granule"

    acc_n = -(-n // _BN) * _BN
    if acc_n == n:
        acc_n += _BN
    pad_rows = acc_n - n

    step = _NW * 16 * _IB
    e2 = -(-e // step) * step
    eblocks = e2 // _IB
    npad = e2 - e
    ar = jnp.arange(npad, dtype=jnp.int32)
    src2 = jnp.concatenate([edge_index[0], ar % n]).reshape(eblocks, _IB)
    dst2 = jnp.concatenate([edge_index[1], n + (ar % pad_rows)]).reshape(eblocks, _IB)

    ones = jnp.ones((_IB,), jnp.float32)
    zeros = jnp.zeros((acc_n,), jnp.float32)
    w2p = jnp.pad(W2, ((0, 0), (0, h - n_cls)))
    b2p = jnp.pad(b2, (0, h - n_cls)).reshape(1, h)

    degs = _deg_kernel(acc_n, eblocks)(dst2, ones, zeros)
    dega = degs[:acc_n].reshape(acc_n, 1)
    degb = degs[acc_n:].reshape(acc_n, 1)

    g1, dinv = _mm1(x, W1, dega, degb, acc_n)

    agg = _agg_kernel(acc_n, eblocks, h)
    accs1 = agg(g1, src2, dst2)
    g2 = _mid(accs1[:acc_n], accs1[acc_n:], g1, dinv, b1.reshape(1, h), w2p,
              acc_n)
    accs2 = agg(g2, src2, dst2)
    return _fin(accs2[:acc_n], accs2[acc_n:], g2, dinv, b2p, n, n_cls, acc_n)
